# baseline (speedup 1.0000x reference)
import functools

import jax
import jax.numpy as jnp
from jax import lax
from jax.experimental import pallas as pl
from jax.experimental.pallas import tpu as pltpu
from jax.experimental.pallas import tpu_sc as plsc

N = 2097152
NUM_BINS = 16
NC = 2
NS = 16
NW = NC * NS
PER_TEC = N // NW
BLK = 4096
NBLK = PER_TEC // BLK
VPB = BLK // 16
HIST = 4 * NUM_BINS * 256
NCOPY = 4
NCOPY1 = 6
EPS = 1e-6
UNROLL = 4

_MESH = plsc.VectorSubcoreMesh(core_axis_name="c", subcore_axis_name="s")
_CP = pltpu.CompilerParams(needs_layout_passes=False)
_MININT = -2147483648


def _srl(x, sh):
    return lax.shift_right_logical(x, jnp.full((16,), sh, jnp.int32))


def _bin_and_keys(p, g):
    y = g * 16.0
    yc = jnp.minimum(jnp.maximum(y, 0.0), 15.0)
    b = yc.astype(jnp.int32)
    valid = (g > 0.0) & (g < 1.0)
    pb = plsc.bitcast(p, jnp.int32)
    kp = jnp.where(pb >= 0, pb | _MININT, ~pb)
    kg = plsc.bitcast(g, jnp.int32)
    return b, valid, kp, kg


def _double_buffered_scan(pred_hbm, gt_hbm, pbuf, gbuf, sems, base0, process,
                          carry_init):
    def start(blk, slot):
        base = base0 + blk * BLK
        pltpu.make_async_copy(
            pred_hbm.at[pl.ds(base, BLK)],
            pbuf.at[pl.ds(slot * BLK, BLK)], sems[slot]).start()
        pltpu.make_async_copy(
            gt_hbm.at[pl.ds(base, BLK)],
            gbuf.at[pl.ds(slot * BLK, BLK)], sems[slot]).start()

    def wait(slot):
        pltpu.make_async_copy(
            pred_hbm.at[pl.ds(0, BLK)],
            pbuf.at[pl.ds(slot * BLK, BLK)], sems[slot]).wait()
        pltpu.make_async_copy(
            gt_hbm.at[pl.ds(0, BLK)],
            gbuf.at[pl.ds(slot * BLK, BLK)], sems[slot]).wait()

    start(0, 0)

    def outer(sb, carry):
        blk0 = sb * 2
        start(blk0 + 1, 1)
        wait(0)
        carry = process(0, carry)

        @pl.when(sb + 1 < NBLK // 2)
        def _():
            start(blk0 + 2, 0)

        wait(1)
        carry = process(1, carry)
        return carry

    return lax.fori_loop(0, NBLK // 2, outer, carry_init)


def _make_pass_kernel(level):
    dig_sh = 32 - 8 * level
    ncopy = NCOPY1 if level == 1 else NCOPY

    @functools.partial(
        pl.kernel,
        out_type=jax.ShapeDtypeStruct((NW * HIST,), jnp.float32),
        mesh=_MESH,
        compiler_params=_CP,
        scratch_types=[
            pltpu.VMEM((ncopy * HIST,), jnp.float32),
            pltpu.VMEM((32,), jnp.int32),
            pltpu.VMEM((2 * BLK,), jnp.float32),
            pltpu.VMEM((2 * BLK,), jnp.float32),
            pltpu.SemaphoreType.DMA,
            pltpu.SemaphoreType.DMA,
        ],
    )
    def pass_kernel(pred_hbm, gt_hbm, pfx_hbm, out_hbm, hist, pfx, pbuf,
                    gbuf, sem0, sem1):
        wid = lax.axis_index("s") * NC + lax.axis_index("c")
        base0 = wid * PER_TEC

        @plsc.parallel_loop(0, ncopy * HIST // 16, step=UNROLL)
        def _zero(i):
            for u in range(UNROLL):
                hist[pl.ds((i + u) * 16, 16)] = jnp.zeros((16,), jnp.float32)

        pltpu.sync_copy(pfx_hbm, pfx)
        lane = lax.iota(jnp.int32, 16)
        if level == 1:
            laneoff = (lane - 6 * _srl(lane * 43691, 18)) * HIST
        else:
            laneoff = (lane & (NCOPY - 1)) * HIST
        ones = jnp.full((16,), 1.0, jnp.float32)

        def one_vec(off):
            p = pbuf[pl.ds(off, 16)]
            g = gbuf[pl.ds(off, 16)]
            b, valid, kp, kg = _bin_and_keys(p, g)
            dp = _srl(kp, dig_sh) & 255
            dg = _srl(kg, dig_sh) & 255
            if level == 1:
                mp = valid
                mg = valid
            else:
                pp = plsc.load_gather(pfx, [b])
                pg = plsc.load_gather(pfx, [b + 16])
                mp = valid & (_srl(kp, dig_sh + 8) == pp)
                mg = valid & (_srl(kg, dig_sh + 8) == pg)
            ip = laneoff + b * 256 + dp
            ig = laneoff + b * 256 + dg
            plsc.addupdate_scatter(hist, [ip], ones, mask=mp)
            plsc.addupdate_scatter(hist, [ip + 4096], p, mask=mp)
            plsc.addupdate_scatter(hist, [ig + 8192], ones, mask=mg)
            plsc.addupdate_scatter(hist, [ig + 12288], g, mask=mg)

        def process(slot, carry):
            @plsc.parallel_loop(0, VPB, step=UNROLL)
            def _body(i):
                for u in range(UNROLL):
                    one_vec(slot * BLK + (i + u) * 16)

            return carry

        _double_buffered_scan(pred_hbm, gt_hbm, pbuf, gbuf, (sem0, sem1),
                              base0, process, 0)

        @plsc.parallel_loop(0, HIST // 16, step=UNROLL)
        def _reduce(i):
            for u in range(UNROLL):
                base = (i + u) * 16
                v = hist[pl.ds(base, 16)]
                for cpy in range(1, ncopy):
                    v = v + hist[pl.ds(cpy * HIST + base, 16)]
                hist[pl.ds(base, 16)] = v

        pltpu.sync_copy(hist.at[pl.ds(0, HIST)],
                        out_hbm.at[pl.ds(wid * HIST, HIST)])

    return pass_kernel


@functools.partial(
    pl.kernel,
    out_type=jax.ShapeDtypeStruct((NW * 16,), jnp.float32),
    mesh=_MESH,
    compiler_params=_CP,
    scratch_types=[
        pltpu.VMEM((80,), jnp.float32),
        pltpu.VMEM((16,), jnp.float32),
        pltpu.VMEM((2 * BLK,), jnp.float32),
        pltpu.VMEM((2 * BLK,), jnp.float32),
        pltpu.SemaphoreType.DMA,
        pltpu.SemaphoreType.DMA,
    ],
)
def _loss_kernel(pred_hbm, gt_hbm, tbl_hbm, out_hbm, tbl, accv, pbuf, gbuf,
                 sem0, sem1):
    wid = lax.axis_index("s") * NC + lax.axis_index("c")
    base0 = wid * PER_TEC
    pltpu.sync_copy(tbl_hbm, tbl)

    def one_vec(off, a):
        p = pbuf[pl.ds(off, 16)]
        g = gbuf[pl.ds(off, 16)]
        y = g * 16.0
        yc = jnp.minimum(jnp.maximum(y, 0.0), 15.0)
        b = yc.astype(jnp.int32)
        valid = (g > 0.0) & (g < 1.0)
        med_p = plsc.load_gather(tbl, [b])
        inv_sp = plsc.load_gather(tbl, [b + 16])
        med_g = plsc.load_gather(tbl, [b + 32])
        inv_sg = plsc.load_gather(tbl, [b + 48])
        w = plsc.load_gather(tbl, [b + 64])
        t = (p - med_p) * inv_sp - (g - med_g) * inv_sg
        return a + jnp.where(valid, jnp.abs(t) * w, 0.0)

    def process(slot, carry):
        @plsc.parallel_loop(0, VPB, step=UNROLL, carry=carry)
        def _body(i, accs):
            return tuple(
                one_vec(slot * BLK + (i + u) * 16, accs[u])
                for u in range(UNROLL))

        return _body

    zero = jnp.zeros((16,), jnp.float32)
    accs = _double_buffered_scan(pred_hbm, gt_hbm, pbuf, gbuf, (sem0, sem1),
                                 base0, process, (zero,) * UNROLL)
    accv[...] = accs[0] + accs[1] + accs[2] + accs[3]
    pltpu.sync_copy(accv, out_hbm.at[pl.ds(wid * 16, 16)])


_PASS_KERNELS = [_make_pass_kernel(level) for level in (1, 2, 3, 4)]


def _select_digit(cnt, vsum, kk, prefix, c_lo, s_lo, c_hi, s_hi):
    c = cnt.astype(jnp.int32)
    csum = jnp.cumsum(c, axis=1)
    ssum = jnp.cumsum(vsum, axis=1)
    sel = jnp.argmax(csum > kk[:, None], axis=1).astype(jnp.int32)
    take = lambda m: jnp.take_along_axis(m, sel[:, None], axis=1)[:, 0]
    cs_at, c_at = take(csum), take(c)
    ss_at, s_at = take(ssum), take(vsum)
    below_c = cs_at - c_at
    below_s = ss_at - s_at
    c_lo = c_lo + below_c
    s_lo = s_lo + below_s
    c_hi = c_hi + (csum[:, -1] - cs_at)
    s_hi = s_hi + (ssum[:, -1] - ss_at)
    kk = kk - below_c
    prefix = (prefix << 8) | sel
    return kk, prefix, c_lo, s_lo, c_hi, s_hi


def kernel(pred_depth, gt_depth, num_bins):
    pred = pred_depth.astype(jnp.float32)
    gt = gt_depth.astype(jnp.float32)

    z16i = jnp.zeros((16,), jnp.int32)
    z16f = jnp.zeros((16,), jnp.float32)
    state = {
        "p": [None, z16i, z16i, z16f, z16i, z16f],
        "g": [None, z16i, z16i, z16f, z16i, z16f],
    }
    n = None
    for lvl in range(4):
        pfx = jnp.concatenate([state["p"][1], state["g"][1]])
        part = _PASS_KERNELS[lvl](pred, gt, pfx)
        H = part.reshape(NW, 4, NUM_BINS, 256).sum(axis=0)
        if lvl == 0:
            n = jnp.sum(H[2], axis=1).astype(jnp.int32)
            k0 = jnp.maximum((n - 1) // 2, 0)
            state["p"][0] = k0
            state["g"][0] = k0
        for a, ci, si in (("p", 0, 1), ("g", 2, 3)):
            st = state[a]
            st[0], st[1], st[2], st[3], st[4], st[5] = _select_digit(
                H[ci], H[si], st[0], st[1], st[2], st[3], st[4], st[5])

    nf = n.astype(jnp.float32)

    def finish(a, key_to_float):
        _, prefix, c_lo, s_lo, c_hi, s_hi = state[a]
        med = key_to_float(prefix)
        s = ((s_hi - med * c_hi.astype(jnp.float32))
             + (med * c_lo.astype(jnp.float32) - s_lo)) / nf
        return med, 1.0 / (s + EPS)

    def pred_key_to_float(key):
        bits = jnp.where(key < 0, key & jnp.int32(0x7FFFFFFF), ~key)
        return lax.bitcast_convert_type(bits, jnp.float32)

    def gt_key_to_float(key):
        return lax.bitcast_convert_type(key, jnp.float32)

    med_p, inv_sp = finish("p", pred_key_to_float)
    med_g, inv_sg = finish("g", gt_key_to_float)
    w = 1.0 / nf

    tbl = jnp.concatenate([med_p, inv_sp, med_g, inv_sg, w])
    partials = _loss_kernel(pred, gt, tbl)
    loss = jnp.sum(partials) / jnp.asarray(num_bins, jnp.float32)
    loss = jnp.where(jnp.any(n == 0), jnp.float32(jnp.nan), loss)
    return loss.astype(pred_depth.dtype)


# speedup vs baseline: 1.0917x; 1.0917x over previous
import functools

import jax
import jax.numpy as jnp
from jax import lax
from jax.experimental import pallas as pl
from jax.experimental.pallas import tpu as pltpu
from jax.experimental.pallas import tpu_sc as plsc

N = 2097152
NUM_BINS = 16
NC = 2
NS = 16
NW = NC * NS
PER_TEC = N // NW
BLK = 4096
NBLK = PER_TEC // BLK
VPB = BLK // 16
HIST = 4 * NUM_BINS * 256
NCOPY = 1
NCOPY1 = 1
EPS = 1e-6
UNROLL = 4

_MESH = plsc.VectorSubcoreMesh(core_axis_name="c", subcore_axis_name="s")
_CP = pltpu.CompilerParams(needs_layout_passes=False)
_MININT = -2147483648


def _srl(x, sh):
    return lax.shift_right_logical(x, jnp.full((16,), sh, jnp.int32))


def _bin_and_keys(p, g):
    y = g * 16.0
    yc = jnp.minimum(jnp.maximum(y, 0.0), 15.0)
    b = yc.astype(jnp.int32)
    valid = (g > 0.0) & (g < 1.0)
    pb = plsc.bitcast(p, jnp.int32)
    kp = jnp.where(pb >= 0, pb | _MININT, ~pb)
    kg = plsc.bitcast(g, jnp.int32)
    return b, valid, kp, kg


def _double_buffered_scan(pred_hbm, gt_hbm, pbuf, gbuf, sems, base0, process,
                          carry_init):
    def start(blk, slot):
        base = base0 + blk * BLK
        pltpu.make_async_copy(
            pred_hbm.at[pl.ds(base, BLK)],
            pbuf.at[pl.ds(slot * BLK, BLK)], sems[slot]).start()
        pltpu.make_async_copy(
            gt_hbm.at[pl.ds(base, BLK)],
            gbuf.at[pl.ds(slot * BLK, BLK)], sems[slot]).start()

    def wait(slot):
        pltpu.make_async_copy(
            pred_hbm.at[pl.ds(0, BLK)],
            pbuf.at[pl.ds(slot * BLK, BLK)], sems[slot]).wait()
        pltpu.make_async_copy(
            gt_hbm.at[pl.ds(0, BLK)],
            gbuf.at[pl.ds(slot * BLK, BLK)], sems[slot]).wait()

    start(0, 0)

    def outer(sb, carry):
        blk0 = sb * 2
        start(blk0 + 1, 1)
        wait(0)
        carry = process(0, carry)

        @pl.when(sb + 1 < NBLK // 2)
        def _():
            start(blk0 + 2, 0)

        wait(1)
        carry = process(1, carry)
        return carry

    return lax.fori_loop(0, NBLK // 2, outer, carry_init)


def _make_pass_kernel(level):
    dig_sh = 32 - 8 * level
    ncopy = NCOPY1 if level == 1 else NCOPY

    @functools.partial(
        pl.kernel,
        out_type=jax.ShapeDtypeStruct((NW * HIST,), jnp.float32),
        mesh=_MESH,
        compiler_params=_CP,
        scratch_types=[
            pltpu.VMEM((ncopy * HIST,), jnp.float32),
            pltpu.VMEM((32,), jnp.int32),
            pltpu.VMEM((2 * BLK,), jnp.float32),
            pltpu.VMEM((2 * BLK,), jnp.float32),
            pltpu.SemaphoreType.DMA,
            pltpu.SemaphoreType.DMA,
        ],
    )
    def pass_kernel(pred_hbm, gt_hbm, pfx_hbm, out_hbm, hist, pfx, pbuf,
                    gbuf, sem0, sem1):
        wid = lax.axis_index("s") * NC + lax.axis_index("c")
        base0 = wid * PER_TEC

        @plsc.parallel_loop(0, ncopy * HIST // 16, step=UNROLL)
        def _zero(i):
            for u in range(UNROLL):
                hist[pl.ds((i + u) * 16, 16)] = jnp.zeros((16,), jnp.float32)

        pltpu.sync_copy(pfx_hbm, pfx)
        laneoff = (lax.iota(jnp.int32, 16) & (ncopy - 1)) * HIST
        ones = jnp.full((16,), 1.0, jnp.float32)

        def one_vec(off):
            p = pbuf[pl.ds(off, 16)]
            g = gbuf[pl.ds(off, 16)]
            b, valid, kp, kg = _bin_and_keys(p, g)
            dp = _srl(kp, dig_sh) & 255
            dg = _srl(kg, dig_sh) & 255
            if level == 1:
                mp = valid
                mg = valid
            else:
                pp = plsc.load_gather(pfx, [b])
                pg = plsc.load_gather(pfx, [b + 16])
                mp = valid & (_srl(kp, dig_sh + 8) == pp)
                mg = valid & (_srl(kg, dig_sh + 8) == pg)
            ip = laneoff + b * 256 + dp
            ig = laneoff + b * 256 + dg
            plsc.addupdate_scatter(hist, [ip], ones, mask=mp)
            plsc.addupdate_scatter(hist, [ip + 4096], p, mask=mp)
            plsc.addupdate_scatter(hist, [ig + 8192], ones, mask=mg)
            plsc.addupdate_scatter(hist, [ig + 12288], g, mask=mg)

        def process(slot, carry):
            @plsc.parallel_loop(0, VPB, step=UNROLL)
            def _body(i):
                for u in range(UNROLL):
                    one_vec(slot * BLK + (i + u) * 16)

            return carry

        _double_buffered_scan(pred_hbm, gt_hbm, pbuf, gbuf, (sem0, sem1),
                              base0, process, 0)

        @plsc.parallel_loop(0, HIST // 16, step=UNROLL)
        def _reduce(i):
            for u in range(UNROLL):
                base = (i + u) * 16
                v = hist[pl.ds(base, 16)]
                for cpy in range(1, ncopy):
                    v = v + hist[pl.ds(cpy * HIST + base, 16)]
                hist[pl.ds(base, 16)] = v

        pltpu.sync_copy(hist.at[pl.ds(0, HIST)],
                        out_hbm.at[pl.ds(wid * HIST, HIST)])

    return pass_kernel


@functools.partial(
    pl.kernel,
    out_type=jax.ShapeDtypeStruct((NW * 16,), jnp.float32),
    mesh=_MESH,
    compiler_params=_CP,
    scratch_types=[
        pltpu.VMEM((80,), jnp.float32),
        pltpu.VMEM((16,), jnp.float32),
        pltpu.VMEM((2 * BLK,), jnp.float32),
        pltpu.VMEM((2 * BLK,), jnp.float32),
        pltpu.SemaphoreType.DMA,
        pltpu.SemaphoreType.DMA,
    ],
)
def _loss_kernel(pred_hbm, gt_hbm, tbl_hbm, out_hbm, tbl, accv, pbuf, gbuf,
                 sem0, sem1):
    wid = lax.axis_index("s") * NC + lax.axis_index("c")
    base0 = wid * PER_TEC
    pltpu.sync_copy(tbl_hbm, tbl)

    def one_vec(off, a):
        p = pbuf[pl.ds(off, 16)]
        g = gbuf[pl.ds(off, 16)]
        y = g * 16.0
        yc = jnp.minimum(jnp.maximum(y, 0.0), 15.0)
        b = yc.astype(jnp.int32)
        valid = (g > 0.0) & (g < 1.0)
        med_p = plsc.load_gather(tbl, [b])
        inv_sp = plsc.load_gather(tbl, [b + 16])
        med_g = plsc.load_gather(tbl, [b + 32])
        inv_sg = plsc.load_gather(tbl, [b + 48])
        w = plsc.load_gather(tbl, [b + 64])
        t = (p - med_p) * inv_sp - (g - med_g) * inv_sg
        return a + jnp.where(valid, jnp.abs(t) * w, 0.0)

    def process(slot, carry):
        @plsc.parallel_loop(0, VPB, step=UNROLL, carry=carry)
        def _body(i, accs):
            return tuple(
                one_vec(slot * BLK + (i + u) * 16, accs[u])
                for u in range(UNROLL))

        return _body

    zero = jnp.zeros((16,), jnp.float32)
    accs = _double_buffered_scan(pred_hbm, gt_hbm, pbuf, gbuf, (sem0, sem1),
                                 base0, process, (zero,) * UNROLL)
    accv[...] = accs[0] + accs[1] + accs[2] + accs[3]
    pltpu.sync_copy(accv, out_hbm.at[pl.ds(wid * 16, 16)])


_PASS_KERNELS = [_make_pass_kernel(level) for level in (1, 2, 3, 4)]


def _select_digit(cnt, vsum, kk, prefix, c_lo, s_lo, c_hi, s_hi):
    c = cnt.astype(jnp.int32)
    csum = jnp.cumsum(c, axis=1)
    ssum = jnp.cumsum(vsum, axis=1)
    sel = jnp.argmax(csum > kk[:, None], axis=1).astype(jnp.int32)
    take = lambda m: jnp.take_along_axis(m, sel[:, None], axis=1)[:, 0]
    cs_at, c_at = take(csum), take(c)
    ss_at, s_at = take(ssum), take(vsum)
    below_c = cs_at - c_at
    below_s = ss_at - s_at
    c_lo = c_lo + below_c
    s_lo = s_lo + below_s
    c_hi = c_hi + (csum[:, -1] - cs_at)
    s_hi = s_hi + (ssum[:, -1] - ss_at)
    kk = kk - below_c
    prefix = (prefix << 8) | sel
    return kk, prefix, c_lo, s_lo, c_hi, s_hi


def kernel(pred_depth, gt_depth, num_bins):
    pred = pred_depth.astype(jnp.float32)
    gt = gt_depth.astype(jnp.float32)

    z16i = jnp.zeros((16,), jnp.int32)
    z16f = jnp.zeros((16,), jnp.float32)
    state = {
        "p": [None, z16i, z16i, z16f, z16i, z16f],
        "g": [None, z16i, z16i, z16f, z16i, z16f],
    }
    n = None
    for lvl in range(4):
        pfx = jnp.concatenate([state["p"][1], state["g"][1]])
        part = _PASS_KERNELS[lvl](pred, gt, pfx)
        H = part.reshape(NW, 4, NUM_BINS, 256).sum(axis=0)
        if lvl == 0:
            n = jnp.sum(H[2], axis=1).astype(jnp.int32)
            k0 = jnp.maximum((n - 1) // 2, 0)
            state["p"][0] = k0
            state["g"][0] = k0
        for a, ci, si in (("p", 0, 1), ("g", 2, 3)):
            st = state[a]
            st[0], st[1], st[2], st[3], st[4], st[5] = _select_digit(
                H[ci], H[si], st[0], st[1], st[2], st[3], st[4], st[5])

    nf = n.astype(jnp.float32)

    def finish(a, key_to_float):
        _, prefix, c_lo, s_lo, c_hi, s_hi = state[a]
        med = key_to_float(prefix)
        s = ((s_hi - med * c_hi.astype(jnp.float32))
             + (med * c_lo.astype(jnp.float32) - s_lo)) / nf
        return med, 1.0 / (s + EPS)

    def pred_key_to_float(key):
        bits = jnp.where(key < 0, key & jnp.int32(0x7FFFFFFF), ~key)
        return lax.bitcast_convert_type(bits, jnp.float32)

    def gt_key_to_float(key):
        return lax.bitcast_convert_type(key, jnp.float32)

    med_p, inv_sp = finish("p", pred_key_to_float)
    med_g, inv_sg = finish("g", gt_key_to_float)
    w = 1.0 / nf

    tbl = jnp.concatenate([med_p, inv_sp, med_g, inv_sg, w])
    partials = _loss_kernel(pred, gt, tbl)
    loss = jnp.sum(partials) / jnp.asarray(num_bins, jnp.float32)
    loss = jnp.where(jnp.any(n == 0), jnp.float32(jnp.nan), loss)
    return loss.astype(pred_depth.dtype)


# speedup vs baseline: 1.0993x; 1.0070x over previous
import functools

import jax
import jax.numpy as jnp
from jax import lax
from jax.experimental import pallas as pl
from jax.experimental.pallas import tpu as pltpu
from jax.experimental.pallas import tpu_sc as plsc

N = 2097152
NUM_BINS = 16
NC = 2
NS = 16
NW = NC * NS
PER_TEC = N // NW
BLK = 4096
NBLK = PER_TEC // BLK
VPB = BLK // 16
HIST = 4 * NUM_BINS * 256
NCOPY = 1
NCOPY1 = 1
EPS = 1e-6
UNROLL = 4

_MESH = plsc.VectorSubcoreMesh(core_axis_name="c", subcore_axis_name="s")
_CP = pltpu.CompilerParams(needs_layout_passes=False)
_MININT = -2147483648


def _srl(x, sh):
    return lax.shift_right_logical(x, jnp.full((16,), sh, jnp.int32))


def _bin_and_keys(p, g):
    y = g * 16.0
    yc = jnp.minimum(jnp.maximum(y, 0.0), 15.0)
    b = yc.astype(jnp.int32)
    valid = (g > 0.0) & (g < 1.0)
    pb = plsc.bitcast(p, jnp.int32)
    kp = jnp.where(pb >= 0, pb | _MININT, ~pb)
    kg = plsc.bitcast(g, jnp.int32)
    return b, valid, kp, kg


def _double_buffered_scan(pred_hbm, gt_hbm, pbuf, gbuf, sems, base0, process,
                          carry_init):
    def start(blk, slot):
        base = base0 + blk * BLK
        pltpu.make_async_copy(
            pred_hbm.at[pl.ds(base, BLK)],
            pbuf.at[pl.ds(slot * BLK, BLK)], sems[slot]).start()
        pltpu.make_async_copy(
            gt_hbm.at[pl.ds(base, BLK)],
            gbuf.at[pl.ds(slot * BLK, BLK)], sems[slot]).start()

    def wait(slot):
        pltpu.make_async_copy(
            pred_hbm.at[pl.ds(0, BLK)],
            pbuf.at[pl.ds(slot * BLK, BLK)], sems[slot]).wait()
        pltpu.make_async_copy(
            gt_hbm.at[pl.ds(0, BLK)],
            gbuf.at[pl.ds(slot * BLK, BLK)], sems[slot]).wait()

    start(0, 0)

    def outer(sb, carry):
        blk0 = sb * 2
        start(blk0 + 1, 1)
        wait(0)
        carry = process(0, carry)

        @pl.when(sb + 1 < NBLK // 2)
        def _():
            start(blk0 + 2, 0)

        wait(1)
        carry = process(1, carry)
        return carry

    return lax.fori_loop(0, NBLK // 2, outer, carry_init)


def _make_pass_kernel(level):
    dig_sh = 32 - 8 * level
    ncopy = NCOPY1 if level == 1 else NCOPY

    @functools.partial(
        pl.kernel,
        out_type=jax.ShapeDtypeStruct((NW * HIST,), jnp.float32),
        mesh=_MESH,
        compiler_params=_CP,
        scratch_types=[
            pltpu.VMEM((ncopy * HIST,), jnp.float32),
            pltpu.VMEM((32,), jnp.int32),
            pltpu.VMEM((2 * BLK,), jnp.float32),
            pltpu.VMEM((2 * BLK,), jnp.float32),
            pltpu.SemaphoreType.DMA,
            pltpu.SemaphoreType.DMA,
        ],
    )
    def pass_kernel(pred_hbm, gt_hbm, pfx_hbm, out_hbm, hist, pfx, pbuf,
                    gbuf, sem0, sem1):
        wid = lax.axis_index("s") * NC + lax.axis_index("c")
        base0 = wid * PER_TEC

        @plsc.parallel_loop(0, ncopy * HIST // 16, step=UNROLL)
        def _zero(i):
            for u in range(UNROLL):
                hist[pl.ds((i + u) * 16, 16)] = jnp.zeros((16,), jnp.float32)

        pltpu.sync_copy(pfx_hbm, pfx)
        laneoff = (lax.iota(jnp.int32, 16) & (ncopy - 1)) * HIST
        ones = jnp.full((16,), 1.0, jnp.float32)

        def one_vec(off):
            p = pbuf[pl.ds(off, 16)]
            g = gbuf[pl.ds(off, 16)]
            b, valid, kp, kg = _bin_and_keys(p, g)
            dp = _srl(kp, dig_sh) & 255
            dg = _srl(kg, dig_sh) & 255
            if level == 1:
                mp = valid
                mg = valid
            else:
                pp = plsc.load_gather(pfx, [b])
                pg = plsc.load_gather(pfx, [b + 16])
                mp = valid & (_srl(kp, dig_sh + 8) == pp)
                mg = valid & (_srl(kg, dig_sh + 8) == pg)
            ip = laneoff + b * 256 + dp
            ig = laneoff + b * 256 + dg
            plsc.addupdate_scatter(hist, [ip], ones, mask=mp)
            plsc.addupdate_scatter(hist, [ip + 4096], p, mask=mp)
            plsc.addupdate_scatter(hist, [ig + 8192], ones, mask=mg)
            plsc.addupdate_scatter(hist, [ig + 12288], g, mask=mg)

        def process(slot, carry):
            @plsc.parallel_loop(0, VPB, step=UNROLL)
            def _body(i):
                for u in range(UNROLL):
                    one_vec(slot * BLK + (i + u) * 16)

            return carry

        _double_buffered_scan(pred_hbm, gt_hbm, pbuf, gbuf, (sem0, sem1),
                              base0, process, 0)

        if ncopy > 1:
            @plsc.parallel_loop(0, HIST // 16, step=UNROLL)
            def _reduce(i):
                for u in range(UNROLL):
                    base = (i + u) * 16
                    v = hist[pl.ds(base, 16)]
                    for cpy in range(1, ncopy):
                        v = v + hist[pl.ds(cpy * HIST + base, 16)]
                    hist[pl.ds(base, 16)] = v

        pltpu.sync_copy(hist.at[pl.ds(0, HIST)],
                        out_hbm.at[pl.ds(wid * HIST, HIST)])

    return pass_kernel


@functools.partial(
    pl.kernel,
    out_type=jax.ShapeDtypeStruct((NW * 16,), jnp.float32),
    mesh=_MESH,
    compiler_params=_CP,
    scratch_types=[
        pltpu.VMEM((64,), jnp.float32),
        pltpu.VMEM((16,), jnp.float32),
        pltpu.VMEM((2 * BLK,), jnp.float32),
        pltpu.VMEM((2 * BLK,), jnp.float32),
        pltpu.SemaphoreType.DMA,
        pltpu.SemaphoreType.DMA,
    ],
)
def _loss_kernel(pred_hbm, gt_hbm, tbl_hbm, out_hbm, tbl, accv, pbuf, gbuf,
                 sem0, sem1):
    wid = lax.axis_index("s") * NC + lax.axis_index("c")
    base0 = wid * PER_TEC
    pltpu.sync_copy(tbl_hbm, tbl)

    def one_vec(off, a):
        p = pbuf[pl.ds(off, 16)]
        g = gbuf[pl.ds(off, 16)]
        y = g * 16.0
        yc = jnp.minimum(jnp.maximum(y, 0.0), 15.0)
        b = yc.astype(jnp.int32)
        valid = (g > 0.0) & (g < 1.0)
        inv_sp = plsc.load_gather(tbl, [b])
        inv_sg = plsc.load_gather(tbl, [b + 16])
        cc = plsc.load_gather(tbl, [b + 32])
        w = plsc.load_gather(tbl, [b + 48])
        t = p * inv_sp - g * inv_sg + cc
        return a + jnp.where(valid, jnp.abs(t) * w, 0.0)

    def process(slot, carry):
        @plsc.parallel_loop(0, VPB, step=UNROLL, carry=carry)
        def _body(i, accs):
            return tuple(
                one_vec(slot * BLK + (i + u) * 16, accs[u])
                for u in range(UNROLL))

        return _body

    zero = jnp.zeros((16,), jnp.float32)
    accs = _double_buffered_scan(pred_hbm, gt_hbm, pbuf, gbuf, (sem0, sem1),
                                 base0, process, (zero,) * UNROLL)
    accv[...] = accs[0] + accs[1] + accs[2] + accs[3]
    pltpu.sync_copy(accv, out_hbm.at[pl.ds(wid * 16, 16)])


_PASS_KERNELS = [_make_pass_kernel(level) for level in (1, 2, 3, 4)]


def _select_digit(cnt, vsum, kk, prefix, c_lo, s_lo, c_hi, s_hi):
    c = cnt.astype(jnp.int32)
    csum = jnp.cumsum(c, axis=1)
    ssum = jnp.cumsum(vsum, axis=1)
    sel = jnp.argmax(csum > kk[:, None], axis=1).astype(jnp.int32)
    take = lambda m: jnp.take_along_axis(m, sel[:, None], axis=1)[:, 0]
    cs_at, c_at = take(csum), take(c)
    ss_at, s_at = take(ssum), take(vsum)
    below_c = cs_at - c_at
    below_s = ss_at - s_at
    c_lo = c_lo + below_c
    s_lo = s_lo + below_s
    c_hi = c_hi + (csum[:, -1] - cs_at)
    s_hi = s_hi + (ssum[:, -1] - ss_at)
    kk = kk - below_c
    prefix = (prefix << 8) | sel
    return kk, prefix, c_lo, s_lo, c_hi, s_hi


def kernel(pred_depth, gt_depth, num_bins):
    pred = pred_depth.astype(jnp.float32)
    gt = gt_depth.astype(jnp.float32)

    z16i = jnp.zeros((16,), jnp.int32)
    z16f = jnp.zeros((16,), jnp.float32)
    state = {
        "p": [None, z16i, z16i, z16f, z16i, z16f],
        "g": [None, z16i, z16i, z16f, z16i, z16f],
    }
    n = None
    for lvl in range(4):
        pfx = jnp.concatenate([state["p"][1], state["g"][1]])
        part = _PASS_KERNELS[lvl](pred, gt, pfx)
        H = part.reshape(NW, 4, NUM_BINS, 256).sum(axis=0)
        if lvl == 0:
            n = jnp.sum(H[2], axis=1).astype(jnp.int32)
            k0 = jnp.maximum((n - 1) // 2, 0)
            state["p"][0] = k0
            state["g"][0] = k0
        for a, ci, si in (("p", 0, 1), ("g", 2, 3)):
            st = state[a]
            st[0], st[1], st[2], st[3], st[4], st[5] = _select_digit(
                H[ci], H[si], st[0], st[1], st[2], st[3], st[4], st[5])

    nf = n.astype(jnp.float32)

    def finish(a, key_to_float):
        _, prefix, c_lo, s_lo, c_hi, s_hi = state[a]
        med = key_to_float(prefix)
        s = ((s_hi - med * c_hi.astype(jnp.float32))
             + (med * c_lo.astype(jnp.float32) - s_lo)) / nf
        return med, 1.0 / (s + EPS)

    def pred_key_to_float(key):
        bits = jnp.where(key < 0, key & jnp.int32(0x7FFFFFFF), ~key)
        return lax.bitcast_convert_type(bits, jnp.float32)

    def gt_key_to_float(key):
        return lax.bitcast_convert_type(key, jnp.float32)

    med_p, inv_sp = finish("p", pred_key_to_float)
    med_g, inv_sg = finish("g", gt_key_to_float)
    w = 1.0 / nf

    tbl = jnp.concatenate(
        [inv_sp, inv_sg, med_g * inv_sg - med_p * inv_sp, w])
    partials = _loss_kernel(pred, gt, tbl)
    loss = jnp.sum(partials) / jnp.asarray(num_bins, jnp.float32)
    loss = jnp.where(jnp.any(n == 0), jnp.float32(jnp.nan), loss)
    return loss.astype(pred_depth.dtype)
